# Initial kernel scaffold; baseline (speedup 1.0000x reference)
import jax
import jax.numpy as jnp
from jax.experimental import pallas as pl


def kernel(x, embeddings):
    raise NotImplementedError("write your pallas kernel here")


# speedup vs baseline: 2.9680x; 2.9680x over previous
import functools

import jax
import jax.numpy as jnp
from jax import lax
from jax.experimental import pallas as pl
from jax.experimental.pallas import tpu as pltpu
from jax.experimental.pallas import tpu_sc as plsc

DIM = 128
NC = 2
NS = 16
NW = NC * NS
CHUNK = 128


def _make_emb_kernel(n_total):
    per_w = n_total // NW
    n_chunks = per_w // CHUNK
    mesh = plsc.VectorSubcoreMesh(core_axis_name="c", subcore_axis_name="s")

    @functools.partial(
        pl.kernel,
        mesh=mesh,
        out_type=jax.ShapeDtypeStruct((n_total, DIM), jnp.float32),
        scratch_types=[
            pltpu.VMEM((per_w,), jnp.int32),
            pltpu.VMEM((CHUNK, DIM), jnp.float32),
            pltpu.SemaphoreType.DMA,
        ],
    )
    def emb(idx_hbm, table_hbm, out_hbm, idx_v, rows_v, sem):
        wid = lax.axis_index("s") * NC + lax.axis_index("c")
        base = wid * per_w
        pltpu.sync_copy(idx_hbm.at[pl.ds(base, per_w)], idx_v)

        def body(i, carry):
            chunk_idx = idx_v.at[pl.ds(i * CHUNK, CHUNK)]
            pltpu.async_copy(table_hbm.at[chunk_idx], rows_v, sem).wait()
            pltpu.sync_copy(rows_v, out_hbm.at[pl.ds(base + i * CHUNK, CHUNK)])
            return carry

        lax.fori_loop(0, n_chunks, body, 0)

    return emb


def kernel(x, embeddings):
    b, l = x.shape
    n_total = b * l
    xf = x.reshape(n_total).astype(jnp.int32)
    out = _make_emb_kernel(n_total)(xf, embeddings)
    return out.reshape(b, l, DIM)


# speedup vs baseline: 3.3360x; 1.1240x over previous
import functools

import jax
import jax.numpy as jnp
from jax import lax
from jax.experimental import pallas as pl
from jax.experimental.pallas import tpu as pltpu
from jax.experimental.pallas import tpu_sc as plsc

DIM = 128
NC = 2
NS = 16
NW = NC * NS
CHUNK = 128


NBUF = 2


def _make_emb_kernel(n_total):
    per_w = n_total // NW
    n_chunks = per_w // CHUNK
    n_groups = n_chunks // NBUF
    mesh = plsc.VectorSubcoreMesh(core_axis_name="c", subcore_axis_name="s")

    @functools.partial(
        pl.kernel,
        mesh=mesh,
        out_type=jax.ShapeDtypeStruct((n_total, DIM), jnp.float32),
        scratch_types=[
            pltpu.VMEM((per_w,), jnp.int32),
            pltpu.VMEM((NBUF, CHUNK, DIM), jnp.float32),
            pltpu.SemaphoreType.DMA((NBUF,)),
        ],
    )
    def emb(idx_hbm, table_hbm, out_hbm, idx_v, rows_v, sems):
        wid = lax.axis_index("s") * NC + lax.axis_index("c")
        base = wid * per_w
        pltpu.sync_copy(idx_hbm.at[pl.ds(base, per_w)], idx_v)

        def gather(c, b):
            return pltpu.make_async_copy(
                table_hbm.at[idx_v.at[pl.ds(c * CHUNK, CHUNK)]],
                rows_v.at[b],
                sems.at[b],
            )

        for b in range(NBUF):
            gather(b, b).start()

        def body(g, carry):
            for b in range(NBUF):
                c = g * NBUF + b
                gather(c, b).wait()
                pltpu.sync_copy(rows_v.at[b], out_hbm.at[pl.ds(base + c * CHUNK, CHUNK)])
                gather(c + NBUF, b).start()
            return carry

        lax.fori_loop(0, n_groups - 1, body, 0)

        for b in range(NBUF):
            c = (n_groups - 1) * NBUF + b
            gather(c, b).wait()
            pltpu.sync_copy(rows_v.at[b], out_hbm.at[pl.ds(base + c * CHUNK, CHUNK)])

    return emb


def kernel(x, embeddings):
    b, l = x.shape
    n_total = b * l
    xf = x.reshape(n_total).astype(jnp.int32)
    out = _make_emb_kernel(n_total)(xf, embeddings)
    return out.reshape(b, l, DIM)


# speedup vs baseline: 3.3399x; 1.0012x over previous
import functools

import jax
import jax.numpy as jnp
from jax import lax
from jax.experimental import pallas as pl
from jax.experimental.pallas import tpu as pltpu
from jax.experimental.pallas import tpu_sc as plsc

DIM = 128
NC = 2
NS = 16
NW = NC * NS
CHUNK = 128


NBUF = 5
KPRE = 3


def _make_emb_kernel(n_total):
    per_w = n_total // NW
    n_chunks = per_w // CHUNK
    n_groups = n_chunks // NBUF
    assert n_chunks % NBUF == 0 and n_groups >= 3
    mesh = plsc.VectorSubcoreMesh(core_axis_name="c", subcore_axis_name="s")

    @functools.partial(
        pl.kernel,
        mesh=mesh,
        out_type=jax.ShapeDtypeStruct((n_total, DIM), jnp.float32),
        scratch_types=[
            pltpu.VMEM((per_w,), jnp.int32),
            pltpu.VMEM((NBUF, CHUNK, DIM), jnp.float32),
            pltpu.SemaphoreType.DMA((NBUF,)),
            pltpu.SemaphoreType.DMA((NBUF,)),
        ],
    )
    def emb(idx_hbm, table_hbm, out_hbm, idx_v, rows_v, gsems, ssems):
        wid = lax.axis_index("s") * NC + lax.axis_index("c")
        base = wid * per_w
        pltpu.sync_copy(idx_hbm.at[pl.ds(base, per_w)], idx_v)

        def gather(c, b):
            return pltpu.make_async_copy(
                table_hbm.at[idx_v.at[pl.ds(c * CHUNK, CHUNK)]],
                rows_v.at[b],
                gsems.at[b],
            )

        def store(c, b):
            return pltpu.make_async_copy(
                rows_v.at[b],
                out_hbm.at[pl.ds(base + c * CHUNK, CHUNK)],
                ssems.at[b],
            )

        for c in range(KPRE):
            gather(c, c).start()

        for k in range(NBUF):
            gather(k, k).wait()
            store(k, k).start()
            b2 = (k + KPRE) % NBUF
            if k + KPRE >= NBUF:
                store(k - (NBUF - KPRE), b2).wait()
            gather(k + KPRE, b2).start()

        def body(g, carry):
            for k in range(NBUF):
                c = g * NBUF + k
                gather(c, k).wait()
                store(c, k).start()
                b2 = (k + KPRE) % NBUF
                store(c - (NBUF - KPRE), b2).wait()
                gather(c + KPRE, b2).start()
            return carry

        lax.fori_loop(1, n_groups - 1, body, 0)

        for k in range(NBUF):
            c = (n_groups - 1) * NBUF + k
            gather(c, k).wait()
            store(c, k).start()
            if k + KPRE < NBUF:
                b2 = (k + KPRE) % NBUF
                store(c - (NBUF - KPRE), b2).wait()
                gather(c + KPRE, b2).start()
        for k in range(NBUF):
            store((n_groups - 1) * NBUF + k, k).wait()

    return emb


def kernel(x, embeddings):
    b, l = x.shape
    n_total = b * l
    xf = x.reshape(n_total).astype(jnp.int32)
    out = _make_emb_kernel(n_total)(xf, embeddings)
    return out.reshape(b, l, DIM)


# speedup vs baseline: 3.3435x; 1.0011x over previous
import functools

import jax
import jax.numpy as jnp
from jax import lax
from jax.experimental import pallas as pl
from jax.experimental.pallas import tpu as pltpu
from jax.experimental.pallas import tpu_sc as plsc

DIM = 128
NC = 2
NS = 16
NW = NC * NS
CHUNK = 128


NBUF = 5
KPRE = 3


def _make_emb_kernel(n_total):
    per_w = n_total // NW
    n_chunks = per_w // CHUNK
    n_groups = n_chunks // NBUF
    assert n_chunks % NBUF == 0 and n_groups >= 3
    mesh = plsc.VectorSubcoreMesh(core_axis_name="c", subcore_axis_name="s")

    @functools.partial(
        pl.kernel,
        mesh=mesh,
        compiler_params=pltpu.CompilerParams(use_tc_tiling_on_sc=True),
        out_type=jax.ShapeDtypeStruct((n_total, DIM), jnp.float32),
        scratch_types=[
            pltpu.VMEM((per_w,), jnp.int32),
            pltpu.VMEM((NBUF, CHUNK, DIM), jnp.float32),
            pltpu.SemaphoreType.DMA((NBUF,)),
            pltpu.SemaphoreType.DMA((NBUF,)),
        ],
    )
    def emb(idx_hbm, table_hbm, out_hbm, idx_v, rows_v, gsems, ssems):
        wid = lax.axis_index("s") * NC + lax.axis_index("c")
        base = wid * per_w
        pltpu.sync_copy(idx_hbm.at[pl.ds(base, per_w)], idx_v)

        def gather(c, b):
            return pltpu.make_async_copy(
                table_hbm.at[idx_v.at[pl.ds(c * CHUNK, CHUNK)]],
                rows_v.at[b],
                gsems.at[b],
            )

        def store(c, b):
            return pltpu.make_async_copy(
                rows_v.at[b],
                out_hbm.at[pl.ds(base + c * CHUNK, CHUNK)],
                ssems.at[b],
            )

        for c in range(KPRE):
            gather(c, c).start()

        for k in range(NBUF):
            gather(k, k).wait()
            store(k, k).start()
            b2 = (k + KPRE) % NBUF
            if k + KPRE >= NBUF:
                store(k - (NBUF - KPRE), b2).wait()
            gather(k + KPRE, b2).start()

        def body(g, carry):
            for k in range(NBUF):
                c = g * NBUF + k
                gather(c, k).wait()
                store(c, k).start()
                b2 = (k + KPRE) % NBUF
                store(c - (NBUF - KPRE), b2).wait()
                gather(c + KPRE, b2).start()
            return carry

        lax.fori_loop(1, n_groups - 1, body, 0)

        for k in range(NBUF):
            c = (n_groups - 1) * NBUF + k
            gather(c, k).wait()
            store(c, k).start()
            if k + KPRE < NBUF:
                b2 = (k + KPRE) % NBUF
                store(c - (NBUF - KPRE), b2).wait()
                gather(c + KPRE, b2).start()
        for k in range(NBUF):
            store((n_groups - 1) * NBUF + k, k).wait()

    return emb


def kernel(x, embeddings):
    b, l = x.shape
    n_total = b * l
    xf = x.reshape(n_total).astype(jnp.int32)
    out = _make_emb_kernel(n_total)(xf, embeddings)
    return out.reshape(b, l, DIM)


# speedup vs baseline: 10.4430x; 3.1234x over previous
import functools

import jax
import jax.numpy as jnp
from jax import lax
from jax.experimental import pallas as pl
from jax.experimental.pallas import tpu as pltpu
from jax.experimental.pallas import tpu_sc as plsc

DIM = 128
NC = 2
NS = 16
NW = NC * NS
CHUNK = 128


NBUF = 5
KPRE = 3


def _make_emb_kernel(n_total):
    per_w = n_total // NW
    n_chunks = per_w // CHUNK
    n_groups = n_chunks // NBUF
    assert n_chunks % NBUF == 0 and n_groups >= 3
    mesh = plsc.VectorSubcoreMesh(core_axis_name="c", subcore_axis_name="s")

    @functools.partial(
        pl.kernel,
        mesh=mesh,
        compiler_params=pltpu.CompilerParams(use_tc_tiling_on_sc=True),
        out_type=jax.ShapeDtypeStruct((n_total, DIM), jnp.float32),
        scratch_types=[
            pltpu.VMEM((per_w,), jnp.int32),
            pltpu.VMEM((NBUF, CHUNK, DIM), jnp.float32),
            pltpu.SemaphoreType.DMA((NBUF,)),
            pltpu.SemaphoreType.DMA((NBUF,)),
        ],
    )
    def emb(idx_hbm, table_hbm, out_hbm, idx_v, rows_v, gsems, ssems):
        wid = lax.axis_index("s") * NC + lax.axis_index("c")
        base = wid * per_w
        pltpu.sync_copy(idx_hbm.at[pl.ds(base, per_w)], idx_v)

        def gather(c, b):
            return pltpu.make_async_copy(
                table_hbm.at[idx_v.at[pl.ds(c * CHUNK, CHUNK)]],
                rows_v.at[b],
                gsems.at[b],
            )

        def store(c, b):
            return pltpu.make_async_copy(
                rows_v.at[b],
                out_hbm.at[pl.ds(base + c * CHUNK, CHUNK)],
                ssems.at[b],
            )

        for c in range(KPRE):
            gather(c, c).start()

        for k in range(NBUF):
            gather(k, k).wait()
            store(k, k).start()
            b2 = (k + KPRE) % NBUF
            if k + KPRE >= NBUF:
                store(k - (NBUF - KPRE), b2).wait()
            gather(k + KPRE, b2).start()

        def body(g, carry):
            for k in range(NBUF):
                c = g * NBUF + k
                gather(c, k).wait()
                store(c, k).start()
                b2 = (k + KPRE) % NBUF
                store(c - (NBUF - KPRE), b2).wait()
                gather(c + KPRE, b2).start()
            return carry

        lax.fori_loop(1, n_groups - 1, body, 0)

        for k in range(NBUF):
            c = (n_groups - 1) * NBUF + k
            gather(c, k).wait()
            store(c, k).start()
            if k + KPRE < NBUF:
                b2 = (k + KPRE) % NBUF
                store(c - (NBUF - KPRE), b2).wait()
                gather(c + KPRE, b2).start()
        for k in range(NBUF):
            store((n_groups - 1) * NBUF + k, k).wait()

    return emb


def kernel(x, embeddings):
    b, l = x.shape
    n_total = b * l
    xf = x.T.reshape(n_total).astype(jnp.int32)
    out = _make_emb_kernel(n_total)(xf, embeddings)
    return out.reshape(l, b, DIM).transpose(1, 0, 2)


# speedup vs baseline: 10.4439x; 1.0001x over previous
import functools

import jax
import jax.numpy as jnp
from jax import lax
from jax.experimental import pallas as pl
from jax.experimental.pallas import tpu as pltpu
from jax.experimental.pallas import tpu_sc as plsc

DIM = 128
NC = 2
NS = 16
NW = NC * NS
CHUNK = 64


NBUF = 10
KPRE = 5


def _make_emb_kernel(n_total):
    per_w = n_total // NW
    n_chunks = per_w // CHUNK
    n_groups = n_chunks // NBUF
    assert n_chunks % NBUF == 0 and n_groups >= 3
    mesh = plsc.VectorSubcoreMesh(core_axis_name="c", subcore_axis_name="s")

    @functools.partial(
        pl.kernel,
        mesh=mesh,
        compiler_params=pltpu.CompilerParams(use_tc_tiling_on_sc=True),
        out_type=jax.ShapeDtypeStruct((n_total, DIM), jnp.float32),
        scratch_types=[
            pltpu.VMEM((per_w,), jnp.int32),
            pltpu.VMEM((NBUF, CHUNK, DIM), jnp.float32),
            pltpu.SemaphoreType.DMA((NBUF,)),
            pltpu.SemaphoreType.DMA((NBUF,)),
        ],
    )
    def emb(idx_hbm, table_hbm, out_hbm, idx_v, rows_v, gsems, ssems):
        wid = lax.axis_index("s") * NC + lax.axis_index("c")
        base = wid * per_w
        pltpu.sync_copy(idx_hbm.at[pl.ds(base, per_w)], idx_v)

        def gather(c, b):
            return pltpu.make_async_copy(
                table_hbm.at[idx_v.at[pl.ds(c * CHUNK, CHUNK)]],
                rows_v.at[b],
                gsems.at[b],
            )

        def store(c, b):
            return pltpu.make_async_copy(
                rows_v.at[b],
                out_hbm.at[pl.ds(base + c * CHUNK, CHUNK)],
                ssems.at[b],
            )

        for c in range(KPRE):
            gather(c, c).start()

        for k in range(NBUF):
            gather(k, k).wait()
            store(k, k).start()
            b2 = (k + KPRE) % NBUF
            if k + KPRE >= NBUF:
                store(k - (NBUF - KPRE), b2).wait()
            gather(k + KPRE, b2).start()

        def body(g, carry):
            for k in range(NBUF):
                c = g * NBUF + k
                gather(c, k).wait()
                store(c, k).start()
                b2 = (k + KPRE) % NBUF
                store(c - (NBUF - KPRE), b2).wait()
                gather(c + KPRE, b2).start()
            return carry

        lax.fori_loop(1, n_groups - 1, body, 0)

        for k in range(NBUF):
            c = (n_groups - 1) * NBUF + k
            gather(c, k).wait()
            store(c, k).start()
            if k + KPRE < NBUF:
                b2 = (k + KPRE) % NBUF
                store(c - (NBUF - KPRE), b2).wait()
                gather(c + KPRE, b2).start()
        for k in range(NBUF):
            store((n_groups - 1) * NBUF + k, k).wait()

    return emb


def kernel(x, embeddings):
    b, l = x.shape
    n_total = b * l
    xf = x.T.reshape(n_total).astype(jnp.int32)
    out = _make_emb_kernel(n_total)(xf, embeddings)
    return out.reshape(l, b, DIM).transpose(1, 0, 2)


# speedup vs baseline: 10.4778x; 1.0032x over previous
import functools

import jax
import jax.numpy as jnp
from jax import lax
from jax.experimental import pallas as pl
from jax.experimental.pallas import tpu as pltpu
from jax.experimental.pallas import tpu_sc as plsc

DIM = 128
NC = 2
NS = 16
NW = NC * NS
CHUNK = 64


NBUF = 10
KPRE = 7


def _make_emb_kernel(n_total):
    per_w = n_total // NW
    n_chunks = per_w // CHUNK
    n_groups = n_chunks // NBUF
    assert n_chunks % NBUF == 0 and n_groups >= 3
    mesh = plsc.VectorSubcoreMesh(core_axis_name="c", subcore_axis_name="s")

    @functools.partial(
        pl.kernel,
        mesh=mesh,
        compiler_params=pltpu.CompilerParams(use_tc_tiling_on_sc=True),
        out_type=jax.ShapeDtypeStruct((n_total, DIM), jnp.float32),
        scratch_types=[
            pltpu.VMEM((per_w,), jnp.int32),
            pltpu.VMEM((NBUF, CHUNK, DIM), jnp.float32),
            pltpu.SemaphoreType.DMA((NBUF,)),
            pltpu.SemaphoreType.DMA((NBUF,)),
        ],
    )
    def emb(idx_hbm, table_hbm, out_hbm, idx_v, rows_v, gsems, ssems):
        wid = lax.axis_index("s") * NC + lax.axis_index("c")
        base = wid * per_w
        pltpu.sync_copy(idx_hbm.at[pl.ds(base, per_w)], idx_v)

        def gather(c, b):
            return pltpu.make_async_copy(
                table_hbm.at[idx_v.at[pl.ds(c * CHUNK, CHUNK)]],
                rows_v.at[b],
                gsems.at[b],
            )

        def store(c, b):
            return pltpu.make_async_copy(
                rows_v.at[b],
                out_hbm.at[pl.ds(base + c * CHUNK, CHUNK)],
                ssems.at[b],
            )

        for c in range(KPRE):
            gather(c, c).start()

        for k in range(NBUF):
            gather(k, k).wait()
            store(k, k).start()
            b2 = (k + KPRE) % NBUF
            if k + KPRE >= NBUF:
                store(k - (NBUF - KPRE), b2).wait()
            gather(k + KPRE, b2).start()

        def body(g, carry):
            for k in range(NBUF):
                c = g * NBUF + k
                gather(c, k).wait()
                store(c, k).start()
                b2 = (k + KPRE) % NBUF
                store(c - (NBUF - KPRE), b2).wait()
                gather(c + KPRE, b2).start()
            return carry

        lax.fori_loop(1, n_groups - 1, body, 0)

        for k in range(NBUF):
            c = (n_groups - 1) * NBUF + k
            gather(c, k).wait()
            store(c, k).start()
            if k + KPRE < NBUF:
                b2 = (k + KPRE) % NBUF
                store(c - (NBUF - KPRE), b2).wait()
                gather(c + KPRE, b2).start()
        for k in range(NBUF):
            store((n_groups - 1) * NBUF + k, k).wait()

    return emb


def kernel(x, embeddings):
    b, l = x.shape
    n_total = b * l
    xf = x.T.reshape(n_total).astype(jnp.int32)
    out = _make_emb_kernel(n_total)(xf, embeddings)
    return out.reshape(l, b, DIM).transpose(1, 0, 2)
